# Initial kernel scaffold; baseline (speedup 1.0000x reference)
import jax
import jax.numpy as jnp
from jax.experimental import pallas as pl


def kernel(x, W1, b1, g1, be1, W2, b2, g2, be2, W3, b3, g3, be3, W4, b4, g4, be4, Wl1, bl1, g5, be5, Wl2, bl2, g6, be6, Wl3, bl3):
    raise NotImplementedError("write your pallas kernel here")


# speedup vs baseline: 5.0495x; 5.0495x over previous
import functools

import jax
import jax.numpy as jnp
from jax.experimental import pallas as pl
from jax.experimental.pallas import tpu as pltpu
from jax.experimental.pallas import tpu_sc as plsc

NPTS = 10000
NPAD = 10240
KNN = 20
EPSBN = 1e-5
BIG = 1e30
RBLK = 256
NWORK = 32
CROWS = 4


def _std_body(x_ref, o_ref):
    x = x_ref[...]
    rid = jax.lax.broadcasted_iota(jnp.int32, x.shape, 0)
    valid = (rid < NPTS).astype(jnp.float32)
    mean = jnp.sum(x * valid, axis=0, keepdims=True) / NPTS
    c = (x - mean) * valid
    std = jnp.sqrt(jnp.sum(c * c, axis=0, keepdims=True) / (NPTS - 1))
    o_ref[...] = c / std


def _standardize(x_pad):
    return pl.pallas_call(
        _std_body,
        out_shape=jax.ShapeDtypeStruct((NPAD, 2), jnp.float32),
    )(x_pad)


def _layer_tc_body(xq_ref, xt_ref, wab_ref, bvec_ref,
                   idx_ref, a_ref, b_ref, *, dout):
    xq = xq_ref[...]
    xt = xt_ref[...]
    ab = jnp.dot(xq, wab_ref[...], precision=jax.lax.Precision.HIGHEST,
                 preferred_element_type=jnp.float32)
    a_ref[...] = ab[:, :dout] + bvec_ref[...]
    b_ref[...] = ab[:, dout:]
    sq_q = jnp.sum(xq * xq, axis=1, keepdims=True)
    sq_a = jnp.sum(xt * xt, axis=0, keepdims=True)
    xx = jnp.dot(xq.astype(jnp.bfloat16), xt.astype(jnp.bfloat16),
                 preferred_element_type=jnp.float32)
    col = jax.lax.broadcasted_iota(jnp.int32, xx.shape, 1)
    dist = sq_q - 2.0 * xx + sq_a + jnp.where(col >= NPTS, BIG, 0.0)
    for kk in range(KNN):
        m = jnp.min(dist, axis=1, keepdims=True)
        eq = dist == m
        j = jnp.min(jnp.where(eq, col, NPAD), axis=1).astype(jnp.int32)
        idx_ref[:, kk] = j
        if kk + 1 < KNN:
            dist = jnp.where(eq, BIG, dist)


def _layer_tc(x_pad, xt, wab, bvec, dout):
    d = x_pad.shape[1]
    nblk = NPAD // RBLK
    return pl.pallas_call(
        functools.partial(_layer_tc_body, dout=dout),
        grid=(nblk,),
        in_specs=[
            pl.BlockSpec((RBLK, d), lambda i: (i, 0)),
            pl.BlockSpec((d, NPAD), lambda i: (0, 0)),
            pl.BlockSpec((d, 2 * dout), lambda i: (0, 0)),
            pl.BlockSpec((1, dout), lambda i: (0, 0)),
        ],
        out_specs=[
            pl.BlockSpec((RBLK, KNN), lambda i: (i, 0)),
            pl.BlockSpec((RBLK, dout), lambda i: (i, 0)),
            pl.BlockSpec((RBLK, dout), lambda i: (i, 0)),
        ],
        out_shape=[
            jax.ShapeDtypeStruct((NPAD, KNN), jnp.int32),
            jax.ShapeDtypeStruct((NPAD, dout), jnp.float32),
            jax.ShapeDtypeStruct((NPAD, dout), jnp.float32),
        ],
    )(x_pad, xt, wab, bvec)


def _gather_max_sc(bmat, idx_flat, amat, svec, bevec, dout):
    rows_per_w = NPAD // NWORK
    nch = rows_per_w // CROWS
    ncols = dout // 16
    mesh = plsc.VectorSubcoreMesh(core_axis_name="c", subcore_axis_name="s")

    @functools.partial(
        pl.kernel, mesh=mesh,
        out_type=jax.ShapeDtypeStruct((NPAD, dout), jnp.float32),
        scratch_types=[
            pltpu.VMEM((rows_per_w * KNN,), jnp.int32),
            pltpu.VMEM((CROWS * KNN, dout), jnp.float32),
            pltpu.VMEM((CROWS * KNN, dout), jnp.float32),
            pltpu.VMEM((CROWS, dout), jnp.float32),
            pltpu.VMEM((CROWS, dout), jnp.float32),
            pltpu.VMEM((CROWS, dout), jnp.float32),
            pltpu.VMEM((CROWS, dout), jnp.float32),
            pltpu.VMEM((dout,), jnp.float32),
            pltpu.VMEM((dout,), jnp.float32),
            pltpu.SemaphoreType.DMA,
            pltpu.SemaphoreType.DMA,
            pltpu.SemaphoreType.DMA,
            pltpu.SemaphoreType.DMA,
            pltpu.SemaphoreType.DMA,
            pltpu.SemaphoreType.DMA,
        ],
    )
    def k(bmat_hbm, idx_hbm, a_hbm, s_hbm, be_hbm, y_hbm,
          idx_v, rows0, rows1, arow0, arow1, yrow0, yrow1, s_v, be_v,
          sg0, sg1, sa0, sa1, sy0, sy1):
        wid = jax.lax.axis_index("s") * 2 + jax.lax.axis_index("c")
        base0 = wid * rows_per_w
        pltpu.sync_copy(s_hbm, s_v)
        pltpu.sync_copy(be_hbm, be_v)
        pltpu.sync_copy(idx_hbm.at[pl.ds(base0 * KNN, rows_per_w * KNN)], idx_v)

        rows = (rows0, rows1)
        arow = (arow0, arow1)
        yrow = (yrow0, yrow1)
        sg = (sg0, sg1)
        sa = (sa0, sa1)
        sy = (sy0, sy1)

        def issue(ch, par):
            pltpu.async_copy(
                bmat_hbm.at[idx_v.at[pl.ds(ch * CROWS * KNN, CROWS * KNN)]],
                rows[par], sg[par])
            pltpu.async_copy(a_hbm.at[pl.ds(base0 + ch * CROWS, CROWS)],
                             arow[par], sa[par])

        issue(0, 0)
        issue(1, 1)

        def pair(p, _):
            for par in range(2):
                ch = 2 * p + par
                rv, av, yv = rows[par], arow[par], yrow[par]
                pltpu.make_async_copy(bmat_hbm.at[idx_v.at[pl.ds(0, CROWS * KNN)]],
                                      rv, sg[par]).wait()
                pltpu.make_async_copy(a_hbm.at[pl.ds(0, CROWS)], av,
                                      sa[par]).wait()

                @pl.when(ch >= 2)
                def _():
                    pltpu.make_async_copy(yv, y_hbm.at[pl.ds(0, CROWS)],
                                          sy[par]).wait()

                def row_body(r, _):
                    for c in range(ncols):
                        sl = pl.ds(c * 16, 16)
                        acc = rv[r * KNN, sl]
                        for kk in range(1, KNN):
                            acc = jnp.maximum(acc, rv[r * KNN + kk, sl])
                        v = jnp.maximum(av[r, sl] + acc, 0.0)
                        v = v * s_v[sl] + be_v[sl]
                        yv[r, sl] = jnp.where(v >= 0, v, 0.2 * v)
                    return ()

                jax.lax.fori_loop(0, CROWS, row_body, (), unroll=True)
                pltpu.async_copy(yv, y_hbm.at[pl.ds(base0 + ch * CROWS, CROWS)],
                                 sy[par])

                @pl.when(ch + 2 < nch)
                def _():
                    issue(ch + 2, par)
            return ()

        jax.lax.fori_loop(0, nch // 2, pair, (), unroll=False)
        pltpu.make_async_copy(yrow0, y_hbm.at[pl.ds(0, CROWS)], sy0).wait()
        pltpu.make_async_copy(yrow1, y_hbm.at[pl.ds(0, CROWS)], sy1).wait()

    return k(bmat, idx_flat, amat, svec, bevec)


def _head_body(xc_ref, w1_ref, b1_ref, w2_ref, b2_ref, w3_ref, b3_ref, o_ref):
    xc = xc_ref[...]
    h = jnp.dot(xc.astype(jnp.bfloat16), w1_ref[...].astype(jnp.bfloat16),
                preferred_element_type=jnp.float32) + b1_ref[0, :512]
    h = h / jnp.sqrt(1.0 + EPSBN) * b1_ref[1, :512] + b1_ref[2, :512]
    h = jnp.where(h >= 0, h, 0.2 * h)
    h = jnp.dot(h.astype(jnp.bfloat16), w2_ref[...].astype(jnp.bfloat16),
                preferred_element_type=jnp.float32) + b2_ref[0, :256]
    h = h / jnp.sqrt(1.0 + EPSBN) * b2_ref[1, :256] + b2_ref[2, :256]
    h = jnp.where(h >= 0, h, 0.2 * h)
    o = jnp.dot(h.astype(jnp.bfloat16), w3_ref[...].astype(jnp.bfloat16),
                preferred_element_type=jnp.float32) + b3_ref[...]
    o_ref[...] = 1.0 / (1.0 + jnp.exp(-o))


def _head(xc, w1, p1, w2, p2, w3, p3):
    nblk = NPAD // 512
    return pl.pallas_call(
        _head_body,
        grid=(nblk,),
        in_specs=[
            pl.BlockSpec((512, 512), lambda i: (i, 0)),
            pl.BlockSpec((512, 512), lambda i: (0, 0)),
            pl.BlockSpec((3, 512), lambda i: (0, 0)),
            pl.BlockSpec((512, 256), lambda i: (0, 0)),
            pl.BlockSpec((3, 256), lambda i: (0, 0)),
            pl.BlockSpec((256, 1), lambda i: (0, 0)),
            pl.BlockSpec((1, 1), lambda i: (0, 0)),
        ],
        out_specs=pl.BlockSpec((512, 1), lambda i: (i, 0)),
        out_shape=jax.ShapeDtypeStruct((NPAD, 1), jnp.float32),
    )(xc, w1, p1, w2, p2, w3, p3)


def _edge_layer(x_pad, W, b, g, be):
    dout = W.shape[1]
    dpad = max(dout, 128)
    if dout < dpad:
        W = jnp.pad(W, ((0, 0), (0, dpad - dout)))
        b = jnp.pad(b, (0, dpad - dout))
        g = jnp.pad(g, (0, dpad - dout))
        be = jnp.pad(be, (0, dpad - dout))
        dout = dpad
    d = x_pad.shape[1]
    din = W.shape[0] // 2
    if d > din:
        W = jnp.concatenate([
            jnp.pad(W[:din], ((0, d - din), (0, 0))),
            jnp.pad(W[din:], ((0, d - din), (0, 0))),
        ], axis=0)
    wab = jnp.concatenate([W[:d] - W[d:], W[d:]], axis=1)
    idx, amat, bmat = _layer_tc(x_pad, x_pad.T, wab, b[None, :], dout)
    svec = g / jnp.sqrt(1.0 + EPSBN)
    y = _gather_max_sc(bmat, idx.reshape(-1), amat, svec, be, dout)
    return y


def kernel(x, W1, b1, g1, be1, W2, b2, g2, be2, W3, b3, g3, be3,
           W4, b4, g4, be4, Wl1, bl1, g5, be5, Wl2, bl2, g6, be6, Wl3, bl3):
    x_pad = jnp.pad(x, ((0, NPAD - NPTS), (0, 0)))
    x0 = _standardize(x_pad)
    x1 = _edge_layer(x0, W1, b1, g1, be1)
    x2 = _edge_layer(x1, W2, b2, g2, be2)
    x3 = _edge_layer(x2, W3, b3, g3, be3)
    x4 = _edge_layer(x3, W4, b4, g4, be4)
    xc = jnp.concatenate([x1[:, :64], x2[:, :64], x3, x4], axis=-1)
    p1 = jnp.stack([bl1, g5, be5])
    p2 = jnp.stack([bl2, g6, be6])
    out = _head(xc, Wl1, p1, Wl2, p2, Wl3, bl3[None, :])
    return out[:NPTS, 0]


# speedup vs baseline: 5.5145x; 1.0921x over previous
import functools

import jax
import jax.numpy as jnp
from jax.experimental import pallas as pl
from jax.experimental.pallas import tpu as pltpu
from jax.experimental.pallas import tpu_sc as plsc

NPTS = 10000
NPAD = 10240
KNN = 20
EPSBN = 1e-5
BIG = 1e30
RBLK = 512
NWORK = 32
CROWS = 4


def _std_body(x_ref, o_ref):
    x = x_ref[...]
    rid = jax.lax.broadcasted_iota(jnp.int32, x.shape, 0)
    valid = (rid < NPTS).astype(jnp.float32)
    mean = jnp.sum(x * valid, axis=0, keepdims=True) / NPTS
    c = (x - mean) * valid
    std = jnp.sqrt(jnp.sum(c * c, axis=0, keepdims=True) / (NPTS - 1))
    o_ref[...] = c / std


def _standardize(x_pad):
    return pl.pallas_call(
        _std_body,
        out_shape=jax.ShapeDtypeStruct((NPAD, 2), jnp.float32),
    )(x_pad)


def _layer_tc_body(xq_ref, xt_ref, wab_ref, bvec_ref,
                   idx_ref, a_ref, b_ref, *, dout):
    xq = xq_ref[...]
    xt = xt_ref[...]
    ab = jnp.dot(xq, wab_ref[...], precision=jax.lax.Precision.HIGHEST,
                 preferred_element_type=jnp.float32)
    a_ref[...] = ab[:, :dout] + bvec_ref[...]
    b_ref[...] = ab[:, dout:]
    sq_q = jnp.sum(xq * xq, axis=1, keepdims=True)
    sq_a = jnp.sum(xt * xt, axis=0, keepdims=True)
    xx = jnp.dot(xq.astype(jnp.bfloat16), xt.astype(jnp.bfloat16),
                 preferred_element_type=jnp.float32)
    col = jax.lax.broadcasted_iota(jnp.int32, xx.shape, 1)
    dist = sq_q - 2.0 * xx + sq_a + jnp.where(col >= NPTS, BIG, 0.0)
    for kk in range(KNN):
        m = jnp.min(dist, axis=1, keepdims=True)
        eq = dist == m
        j = jnp.min(jnp.where(eq, col, NPAD), axis=1).astype(jnp.int32)
        idx_ref[:, kk] = j
        if kk + 1 < KNN:
            dist = jnp.where(eq, BIG, dist)


def _layer_tc(x_pad, xt, wab, bvec, dout):
    d = x_pad.shape[1]
    nblk = NPAD // RBLK
    return pl.pallas_call(
        functools.partial(_layer_tc_body, dout=dout),
        grid=(nblk,),
        in_specs=[
            pl.BlockSpec((RBLK, d), lambda i: (i, 0)),
            pl.BlockSpec((d, NPAD), lambda i: (0, 0)),
            pl.BlockSpec((d, 2 * dout), lambda i: (0, 0)),
            pl.BlockSpec((1, dout), lambda i: (0, 0)),
        ],
        out_specs=[
            pl.BlockSpec((RBLK, KNN), lambda i: (i, 0)),
            pl.BlockSpec((RBLK, dout), lambda i: (i, 0)),
            pl.BlockSpec((RBLK, dout), lambda i: (i, 0)),
        ],
        out_shape=[
            jax.ShapeDtypeStruct((NPAD, KNN), jnp.int32),
            jax.ShapeDtypeStruct((NPAD, dout), jnp.float32),
            jax.ShapeDtypeStruct((NPAD, dout), jnp.float32),
        ],
    )(x_pad, xt, wab, bvec)


def _gather_max_sc(bmat, idx_flat, amat, svec, bevec, dout):
    rows_per_w = NPAD // NWORK
    nch = rows_per_w // CROWS
    ncols = dout // 16
    mesh = plsc.VectorSubcoreMesh(core_axis_name="c", subcore_axis_name="s")

    @functools.partial(
        pl.kernel, mesh=mesh,
        out_type=jax.ShapeDtypeStruct((NPAD, dout), jnp.float32),
        scratch_types=[
            pltpu.VMEM((rows_per_w * KNN,), jnp.int32),
            pltpu.VMEM((CROWS * KNN, dout), jnp.float32),
            pltpu.VMEM((CROWS * KNN, dout), jnp.float32),
            pltpu.VMEM((CROWS, dout), jnp.float32),
            pltpu.VMEM((CROWS, dout), jnp.float32),
            pltpu.VMEM((CROWS, dout), jnp.float32),
            pltpu.VMEM((CROWS, dout), jnp.float32),
            pltpu.VMEM((dout,), jnp.float32),
            pltpu.VMEM((dout,), jnp.float32),
            pltpu.SemaphoreType.DMA,
            pltpu.SemaphoreType.DMA,
            pltpu.SemaphoreType.DMA,
            pltpu.SemaphoreType.DMA,
            pltpu.SemaphoreType.DMA,
            pltpu.SemaphoreType.DMA,
        ],
    )
    def k(bmat_hbm, idx_hbm, a_hbm, s_hbm, be_hbm, y_hbm,
          idx_v, rows0, rows1, arow0, arow1, yrow0, yrow1, s_v, be_v,
          sg0, sg1, sa0, sa1, sy0, sy1):
        wid = jax.lax.axis_index("s") * 2 + jax.lax.axis_index("c")
        base0 = wid * rows_per_w
        pltpu.sync_copy(s_hbm, s_v)
        pltpu.sync_copy(be_hbm, be_v)
        pltpu.sync_copy(idx_hbm.at[pl.ds(base0 * KNN, rows_per_w * KNN)], idx_v)

        rows = (rows0, rows1)
        arow = (arow0, arow1)
        yrow = (yrow0, yrow1)
        sg = (sg0, sg1)
        sa = (sa0, sa1)
        sy = (sy0, sy1)

        def issue(ch, par):
            pltpu.async_copy(
                bmat_hbm.at[idx_v.at[pl.ds(ch * CROWS * KNN, CROWS * KNN)]],
                rows[par], sg[par])
            pltpu.async_copy(a_hbm.at[pl.ds(base0 + ch * CROWS, CROWS)],
                             arow[par], sa[par])

        issue(0, 0)
        issue(1, 1)

        def pair(p, _):
            for par in range(2):
                ch = 2 * p + par
                rv, av, yv = rows[par], arow[par], yrow[par]
                pltpu.make_async_copy(bmat_hbm.at[idx_v.at[pl.ds(0, CROWS * KNN)]],
                                      rv, sg[par]).wait()
                pltpu.make_async_copy(a_hbm.at[pl.ds(0, CROWS)], av,
                                      sa[par]).wait()

                @pl.when(ch >= 2)
                def _():
                    pltpu.make_async_copy(yv, y_hbm.at[pl.ds(0, CROWS)],
                                          sy[par]).wait()

                def row_body(r, _):
                    for c in range(ncols):
                        sl = pl.ds(c * 16, 16)
                        acc = rv[r * KNN, sl]
                        for kk in range(1, KNN):
                            acc = jnp.maximum(acc, rv[r * KNN + kk, sl])
                        v = jnp.maximum(av[r, sl] + acc, 0.0)
                        v = v * s_v[sl] + be_v[sl]
                        yv[r, sl] = jnp.where(v >= 0, v, 0.2 * v)
                    return ()

                jax.lax.fori_loop(0, CROWS, row_body, (), unroll=True)
                pltpu.async_copy(yv, y_hbm.at[pl.ds(base0 + ch * CROWS, CROWS)],
                                 sy[par])

                @pl.when(ch + 2 < nch)
                def _():
                    issue(ch + 2, par)
            return ()

        jax.lax.fori_loop(0, nch // 2, pair, (), unroll=False)
        pltpu.make_async_copy(yrow0, y_hbm.at[pl.ds(0, CROWS)], sy0).wait()
        pltpu.make_async_copy(yrow1, y_hbm.at[pl.ds(0, CROWS)], sy1).wait()

    return k(bmat, idx_flat, amat, svec, bevec)


def _head_body(xc_ref, w1_ref, b1_ref, w2_ref, b2_ref, w3_ref, b3_ref, o_ref):
    xc = xc_ref[...]
    h = jnp.dot(xc.astype(jnp.bfloat16), w1_ref[...].astype(jnp.bfloat16),
                preferred_element_type=jnp.float32) + b1_ref[0, :512]
    h = h / jnp.sqrt(1.0 + EPSBN) * b1_ref[1, :512] + b1_ref[2, :512]
    h = jnp.where(h >= 0, h, 0.2 * h)
    h = jnp.dot(h.astype(jnp.bfloat16), w2_ref[...].astype(jnp.bfloat16),
                preferred_element_type=jnp.float32) + b2_ref[0, :256]
    h = h / jnp.sqrt(1.0 + EPSBN) * b2_ref[1, :256] + b2_ref[2, :256]
    h = jnp.where(h >= 0, h, 0.2 * h)
    o = jnp.dot(h.astype(jnp.bfloat16), w3_ref[...].astype(jnp.bfloat16),
                preferred_element_type=jnp.float32) + b3_ref[...]
    o_ref[...] = 1.0 / (1.0 + jnp.exp(-o))


def _head(xc, w1, p1, w2, p2, w3, p3):
    nblk = NPAD // 512
    return pl.pallas_call(
        _head_body,
        grid=(nblk,),
        in_specs=[
            pl.BlockSpec((512, 512), lambda i: (i, 0)),
            pl.BlockSpec((512, 512), lambda i: (0, 0)),
            pl.BlockSpec((3, 512), lambda i: (0, 0)),
            pl.BlockSpec((512, 256), lambda i: (0, 0)),
            pl.BlockSpec((3, 256), lambda i: (0, 0)),
            pl.BlockSpec((256, 1), lambda i: (0, 0)),
            pl.BlockSpec((1, 1), lambda i: (0, 0)),
        ],
        out_specs=pl.BlockSpec((512, 1), lambda i: (i, 0)),
        out_shape=jax.ShapeDtypeStruct((NPAD, 1), jnp.float32),
    )(xc, w1, p1, w2, p2, w3, p3)


def _edge_layer(x_pad, W, b, g, be):
    dout = W.shape[1]
    dpad = max(dout, 128)
    if dout < dpad:
        W = jnp.pad(W, ((0, 0), (0, dpad - dout)))
        b = jnp.pad(b, (0, dpad - dout))
        g = jnp.pad(g, (0, dpad - dout))
        be = jnp.pad(be, (0, dpad - dout))
        dout = dpad
    d = x_pad.shape[1]
    din = W.shape[0] // 2
    if d > din:
        W = jnp.concatenate([
            jnp.pad(W[:din], ((0, d - din), (0, 0))),
            jnp.pad(W[din:], ((0, d - din), (0, 0))),
        ], axis=0)
    wab = jnp.concatenate([W[:d] - W[d:], W[d:]], axis=1)
    idx, amat, bmat = _layer_tc(x_pad, x_pad.T, wab, b[None, :], dout)
    svec = g / jnp.sqrt(1.0 + EPSBN)
    y = _gather_max_sc(bmat, idx.reshape(-1), amat, svec, be, dout)
    return y


def kernel(x, W1, b1, g1, be1, W2, b2, g2, be2, W3, b3, g3, be3,
           W4, b4, g4, be4, Wl1, bl1, g5, be5, Wl2, bl2, g6, be6, Wl3, bl3):
    x_pad = jnp.pad(x, ((0, NPAD - NPTS), (0, 0)))
    x0 = _standardize(x_pad)
    x1 = _edge_layer(x0, W1, b1, g1, be1)
    x2 = _edge_layer(x1, W2, b2, g2, be2)
    x3 = _edge_layer(x2, W3, b3, g3, be3)
    x4 = _edge_layer(x3, W4, b4, g4, be4)
    xc = jnp.concatenate([x1[:, :64], x2[:, :64], x3, x4], axis=-1)
    p1 = jnp.stack([bl1, g5, be5])
    p2 = jnp.stack([bl2, g6, be6])
    out = _head(xc, Wl1, p1, Wl2, p2, Wl3, bl3[None, :])
    return out[:NPTS, 0]


# speedup vs baseline: 5.6005x; 1.0156x over previous
import functools

import jax
import jax.numpy as jnp
from jax.experimental import pallas as pl
from jax.experimental.pallas import tpu as pltpu
from jax.experimental.pallas import tpu_sc as plsc

NPTS = 10000
NPAD = 10240
KNN = 20
EPSBN = 1e-5
BIG = 1e30
RBLK = 640
NWORK = 32
CROWS = 4


def _std_body(x_ref, o_ref):
    x = x_ref[...]
    rid = jax.lax.broadcasted_iota(jnp.int32, x.shape, 0)
    valid = (rid < NPTS).astype(jnp.float32)
    mean = jnp.sum(x * valid, axis=0, keepdims=True) / NPTS
    c = (x - mean) * valid
    std = jnp.sqrt(jnp.sum(c * c, axis=0, keepdims=True) / (NPTS - 1))
    o_ref[...] = c / std


def _standardize(x_pad):
    return pl.pallas_call(
        _std_body,
        out_shape=jax.ShapeDtypeStruct((NPAD, 2), jnp.float32),
    )(x_pad)


def _layer_tc_body(xq_ref, xt_ref, wab_ref, bvec_ref,
                   idx_ref, a_ref, b_ref, *, dout):
    xq = xq_ref[...]
    xt = xt_ref[...]
    ab = jnp.dot(xq, wab_ref[...], precision=jax.lax.Precision.HIGHEST,
                 preferred_element_type=jnp.float32)
    a_ref[...] = ab[:, :dout] + bvec_ref[...]
    b_ref[...] = ab[:, dout:]
    sq_q = jnp.sum(xq * xq, axis=1, keepdims=True)
    sq_a = jnp.sum(xt * xt, axis=0, keepdims=True)
    xx = jnp.dot(xq.astype(jnp.bfloat16), xt.astype(jnp.bfloat16),
                 preferred_element_type=jnp.float32)
    col = jax.lax.broadcasted_iota(jnp.int32, xx.shape, 1)
    dist = sq_q - 2.0 * xx + sq_a + jnp.where(col >= NPTS, BIG, 0.0)
    for kk in range(KNN):
        m = jnp.min(dist, axis=1, keepdims=True)
        eq = dist == m
        j = jnp.min(jnp.where(eq, col, NPAD), axis=1).astype(jnp.int32)
        idx_ref[:, kk] = j
        if kk + 1 < KNN:
            dist = jnp.where(eq, BIG, dist)


def _layer_tc(x_pad, xt, wab, bvec, dout):
    d = x_pad.shape[1]
    nblk = NPAD // RBLK
    return pl.pallas_call(
        functools.partial(_layer_tc_body, dout=dout),
        grid=(nblk,),
        in_specs=[
            pl.BlockSpec((RBLK, d), lambda i: (i, 0)),
            pl.BlockSpec((d, NPAD), lambda i: (0, 0)),
            pl.BlockSpec((d, 2 * dout), lambda i: (0, 0)),
            pl.BlockSpec((1, dout), lambda i: (0, 0)),
        ],
        out_specs=[
            pl.BlockSpec((RBLK, KNN), lambda i: (i, 0)),
            pl.BlockSpec((RBLK, dout), lambda i: (i, 0)),
            pl.BlockSpec((RBLK, dout), lambda i: (i, 0)),
        ],
        out_shape=[
            jax.ShapeDtypeStruct((NPAD, KNN), jnp.int32),
            jax.ShapeDtypeStruct((NPAD, dout), jnp.float32),
            jax.ShapeDtypeStruct((NPAD, dout), jnp.float32),
        ],
    )(x_pad, xt, wab, bvec)


def _gather_max_sc(bmat, idx_flat, amat, svec, bevec, dout):
    rows_per_w = NPAD // NWORK
    nch = rows_per_w // CROWS
    ncols = dout // 16
    mesh = plsc.VectorSubcoreMesh(core_axis_name="c", subcore_axis_name="s")

    @functools.partial(
        pl.kernel, mesh=mesh,
        out_type=jax.ShapeDtypeStruct((NPAD, dout), jnp.float32),
        scratch_types=[
            pltpu.VMEM((rows_per_w * KNN,), jnp.int32),
            pltpu.VMEM((CROWS * KNN, dout), jnp.float32),
            pltpu.VMEM((CROWS * KNN, dout), jnp.float32),
            pltpu.VMEM((CROWS, dout), jnp.float32),
            pltpu.VMEM((CROWS, dout), jnp.float32),
            pltpu.VMEM((CROWS, dout), jnp.float32),
            pltpu.VMEM((CROWS, dout), jnp.float32),
            pltpu.VMEM((dout,), jnp.float32),
            pltpu.VMEM((dout,), jnp.float32),
            pltpu.SemaphoreType.DMA,
            pltpu.SemaphoreType.DMA,
            pltpu.SemaphoreType.DMA,
            pltpu.SemaphoreType.DMA,
            pltpu.SemaphoreType.DMA,
            pltpu.SemaphoreType.DMA,
        ],
    )
    def k(bmat_hbm, idx_hbm, a_hbm, s_hbm, be_hbm, y_hbm,
          idx_v, rows0, rows1, arow0, arow1, yrow0, yrow1, s_v, be_v,
          sg0, sg1, sa0, sa1, sy0, sy1):
        wid = jax.lax.axis_index("s") * 2 + jax.lax.axis_index("c")
        base0 = wid * rows_per_w
        pltpu.sync_copy(s_hbm, s_v)
        pltpu.sync_copy(be_hbm, be_v)
        pltpu.sync_copy(idx_hbm.at[pl.ds(base0 * KNN, rows_per_w * KNN)], idx_v)

        rows = (rows0, rows1)
        arow = (arow0, arow1)
        yrow = (yrow0, yrow1)
        sg = (sg0, sg1)
        sa = (sa0, sa1)
        sy = (sy0, sy1)

        def issue(ch, par):
            pltpu.async_copy(
                bmat_hbm.at[idx_v.at[pl.ds(ch * CROWS * KNN, CROWS * KNN)]],
                rows[par], sg[par])
            pltpu.async_copy(a_hbm.at[pl.ds(base0 + ch * CROWS, CROWS)],
                             arow[par], sa[par])

        issue(0, 0)
        issue(1, 1)

        def pair(p, _):
            for par in range(2):
                ch = 2 * p + par
                rv, av, yv = rows[par], arow[par], yrow[par]
                pltpu.make_async_copy(bmat_hbm.at[idx_v.at[pl.ds(0, CROWS * KNN)]],
                                      rv, sg[par]).wait()
                pltpu.make_async_copy(a_hbm.at[pl.ds(0, CROWS)], av,
                                      sa[par]).wait()

                @pl.when(ch >= 2)
                def _():
                    pltpu.make_async_copy(yv, y_hbm.at[pl.ds(0, CROWS)],
                                          sy[par]).wait()

                def row_body(r, _):
                    for c in range(ncols):
                        sl = pl.ds(c * 16, 16)
                        acc = rv[r * KNN, sl]
                        for kk in range(1, KNN):
                            acc = jnp.maximum(acc, rv[r * KNN + kk, sl])
                        v = jnp.maximum(av[r, sl] + acc, 0.0)
                        v = v * s_v[sl] + be_v[sl]
                        yv[r, sl] = jnp.where(v >= 0, v, 0.2 * v)
                    return ()

                jax.lax.fori_loop(0, CROWS, row_body, (), unroll=True)
                pltpu.async_copy(yv, y_hbm.at[pl.ds(base0 + ch * CROWS, CROWS)],
                                 sy[par])

                @pl.when(ch + 2 < nch)
                def _():
                    issue(ch + 2, par)
            return ()

        jax.lax.fori_loop(0, nch // 2, pair, (), unroll=False)
        pltpu.make_async_copy(yrow0, y_hbm.at[pl.ds(0, CROWS)], sy0).wait()
        pltpu.make_async_copy(yrow1, y_hbm.at[pl.ds(0, CROWS)], sy1).wait()

    return k(bmat, idx_flat, amat, svec, bevec)


def _head_body(xc_ref, w1_ref, b1_ref, w2_ref, b2_ref, w3_ref, b3_ref, o_ref):
    xc = xc_ref[...]
    h = jnp.dot(xc.astype(jnp.bfloat16), w1_ref[...].astype(jnp.bfloat16),
                preferred_element_type=jnp.float32) + b1_ref[0, :512]
    h = h / jnp.sqrt(1.0 + EPSBN) * b1_ref[1, :512] + b1_ref[2, :512]
    h = jnp.where(h >= 0, h, 0.2 * h)
    h = jnp.dot(h.astype(jnp.bfloat16), w2_ref[...].astype(jnp.bfloat16),
                preferred_element_type=jnp.float32) + b2_ref[0, :256]
    h = h / jnp.sqrt(1.0 + EPSBN) * b2_ref[1, :256] + b2_ref[2, :256]
    h = jnp.where(h >= 0, h, 0.2 * h)
    o = jnp.dot(h.astype(jnp.bfloat16), w3_ref[...].astype(jnp.bfloat16),
                preferred_element_type=jnp.float32) + b3_ref[...]
    o_ref[...] = 1.0 / (1.0 + jnp.exp(-o))


def _head(xc, w1, p1, w2, p2, w3, p3):
    nblk = NPAD // 512
    return pl.pallas_call(
        _head_body,
        grid=(nblk,),
        in_specs=[
            pl.BlockSpec((512, 512), lambda i: (i, 0)),
            pl.BlockSpec((512, 512), lambda i: (0, 0)),
            pl.BlockSpec((3, 512), lambda i: (0, 0)),
            pl.BlockSpec((512, 256), lambda i: (0, 0)),
            pl.BlockSpec((3, 256), lambda i: (0, 0)),
            pl.BlockSpec((256, 1), lambda i: (0, 0)),
            pl.BlockSpec((1, 1), lambda i: (0, 0)),
        ],
        out_specs=pl.BlockSpec((512, 1), lambda i: (i, 0)),
        out_shape=jax.ShapeDtypeStruct((NPAD, 1), jnp.float32),
    )(xc, w1, p1, w2, p2, w3, p3)


def _edge_layer(x_pad, W, b, g, be):
    dout = W.shape[1]
    dpad = max(dout, 128)
    if dout < dpad:
        W = jnp.pad(W, ((0, 0), (0, dpad - dout)))
        b = jnp.pad(b, (0, dpad - dout))
        g = jnp.pad(g, (0, dpad - dout))
        be = jnp.pad(be, (0, dpad - dout))
        dout = dpad
    d = x_pad.shape[1]
    din = W.shape[0] // 2
    if d > din:
        W = jnp.concatenate([
            jnp.pad(W[:din], ((0, d - din), (0, 0))),
            jnp.pad(W[din:], ((0, d - din), (0, 0))),
        ], axis=0)
    wab = jnp.concatenate([W[:d] - W[d:], W[d:]], axis=1)
    idx, amat, bmat = _layer_tc(x_pad, x_pad.T, wab, b[None, :], dout)
    svec = g / jnp.sqrt(1.0 + EPSBN)
    y = _gather_max_sc(bmat, idx.reshape(-1), amat, svec, be, dout)
    return y


def kernel(x, W1, b1, g1, be1, W2, b2, g2, be2, W3, b3, g3, be3,
           W4, b4, g4, be4, Wl1, bl1, g5, be5, Wl2, bl2, g6, be6, Wl3, bl3):
    x_pad = jnp.pad(x, ((0, NPAD - NPTS), (0, 0)))
    x0 = _standardize(x_pad)
    x1 = _edge_layer(x0, W1, b1, g1, be1)
    x2 = _edge_layer(x1, W2, b2, g2, be2)
    x3 = _edge_layer(x2, W3, b3, g3, be3)
    x4 = _edge_layer(x3, W4, b4, g4, be4)
    xc = jnp.concatenate([x1[:, :64], x2[:, :64], x3, x4], axis=-1)
    p1 = jnp.stack([bl1, g5, be5])
    p2 = jnp.stack([bl2, g6, be6])
    out = _head(xc, Wl1, p1, Wl2, p2, Wl3, bl3[None, :])
    return out[:NPTS, 0]
